# Initial kernel scaffold; baseline (speedup 1.0000x reference)
import jax
import jax.numpy as jnp
from jax.experimental import pallas as pl


def kernel(features, edge_index, edge_type, W_in, b_in, comp, bases, loop_w, conv_b, fc_w, fc_b):
    raise NotImplementedError("write your pallas kernel here")


# speedup vs baseline: 36.0972x; 36.0972x over previous
import functools

import jax
import jax.numpy as jnp
from jax import lax
from jax.experimental import pallas as pl
from jax.experimental.pallas import tpu as pltpu
from jax.experimental.pallas import tpu_sc as plsc

N = 10000
E = 320000
H = 128
NC = 2
NS = 16
NW = NC * NS
EPW = E // NW
ITERS = EPW // 16


def _tc_tables(f_ref, m_ref, feat_ref, fcb_ref, tabs_ref, dense_ref):
    g = jnp.dot(f_ref[...], m_ref[...], preferred_element_type=jnp.float32)
    tabs_ref[...] = g[:, 0:6]
    dense = (jnp.sum(feat_ref[...] * g[:, 6:8])
             + jnp.sum(g[:, 8:10]) + fcb_ref[0, 0])
    dense_ref[...] = jnp.reshape(dense, (1, 1))


_sc_mesh = plsc.VectorSubcoreMesh(core_axis_name="c", subcore_axis_name="s")


@functools.partial(
    pl.kernel,
    out_type=jax.ShapeDtypeStruct((NW, 16), jnp.float32),
    mesh=_sc_mesh,
    compiler_params=pltpu.CompilerParams(needs_layout_passes=False),
    scratch_types=[
        pltpu.VMEM((N * 6,), jnp.float32),
        pltpu.VMEM((N * 2,), jnp.float32),
        pltpu.VMEM((16,), jnp.float32),
        pltpu.VMEM((EPW,), jnp.int32),
        pltpu.VMEM((EPW,), jnp.int32),
        pltpu.VMEM((EPW,), jnp.int32),
        pltpu.VMEM((16,), jnp.float32),
    ],
)
def _sc_edges(tabs_hbm, feat_hbm, comp_hbm, src_hbm, dst_hbm, typ_hbm,
              out_hbm, tabs_v, feat_v, comp_v, src_v, dst_v, typ_v, out_v):
    wid = lax.axis_index("s") * NC + lax.axis_index("c")
    base = wid * EPW
    pltpu.sync_copy(tabs_hbm, tabs_v)
    pltpu.sync_copy(feat_hbm, feat_v)
    pltpu.sync_copy(comp_hbm, comp_v)
    pltpu.sync_copy(src_hbm.at[pl.ds(base, EPW)], src_v)
    pltpu.sync_copy(dst_hbm.at[pl.ds(base, EPW)], dst_v)
    pltpu.sync_copy(typ_hbm.at[pl.ds(base, EPW)], typ_v)

    def body(i, acc):
        s2 = src_v[pl.ds(i * 16, 16)] * 2
        d6 = dst_v[pl.ds(i * 16, 16)] * 6
        t2 = typ_v[pl.ds(i * 16, 16)] * 2
        fx = plsc.load_gather(feat_v, [s2])
        fy = plsc.load_gather(feat_v, [s2 + 1])
        p00 = plsc.load_gather(tabs_v, [d6])
        p01 = plsc.load_gather(tabs_v, [d6 + 1])
        p02 = plsc.load_gather(tabs_v, [d6 + 2])
        p10 = plsc.load_gather(tabs_v, [d6 + 3])
        p11 = plsc.load_gather(tabs_v, [d6 + 4])
        p12 = plsc.load_gather(tabs_v, [d6 + 5])
        c0 = plsc.load_gather(comp_v, [t2])
        c1 = plsc.load_gather(comp_v, [t2 + 1])
        e = c0 * (fx * p00 + fy * p01 + p02) + c1 * (fx * p10 + fy * p11 + p12)
        return acc + e

    out_v[...] = lax.fori_loop(0, ITERS, body, jnp.zeros((16,), jnp.float32))
    pltpu.sync_copy(out_v, out_hbm.at[wid])


def kernel(features, edge_index, edge_type, W_in, b_in, comp, bases,
           loop_w, conv_b, fc_w, fc_b):
    F = fc_w.reshape(N, H)
    w_aug = jnp.concatenate([W_in, b_in[None]], axis=0)
    cb0 = w_aug @ bases[0]
    cb1 = w_aug @ bases[1]
    lw = w_aug @ loop_w
    m16 = jnp.concatenate(
        [cb0.T, cb1.T, lw.T, conv_b[:, None], jnp.zeros((H, 6), jnp.float32)],
        axis=1)

    tabs, dense = pl.pallas_call(
        _tc_tables,
        out_shape=[
            jax.ShapeDtypeStruct((N, 6), jnp.float32),
            jax.ShapeDtypeStruct((1, 1), jnp.float32),
        ],
    )(F, m16, features, fc_b.reshape(1, 1))

    src = edge_index[0]
    dst = edge_index[1]
    partials = _sc_edges(tabs.reshape(N * 6), features.reshape(N * 2),
                         comp.reshape(16), src, dst, edge_type)
    total = jnp.sum(partials) + dense[0, 0]
    return jax.nn.sigmoid(total).reshape(1, 1)


# speedup vs baseline: 39.5838x; 1.0966x over previous
import functools

import jax
import jax.numpy as jnp
from jax import lax
from jax.experimental import pallas as pl
from jax.experimental.pallas import tpu as pltpu
from jax.experimental.pallas import tpu_sc as plsc

N = 10000
E = 320000
H = 128
NC = 2
NS = 16
NW = NC * NS
EPW = E // NW
ITERS = EPW // 16


def _pack(a, b):
    ba = lax.bitcast_convert_type(a.astype(jnp.bfloat16), jnp.uint16)
    bb = lax.bitcast_convert_type(b.astype(jnp.bfloat16), jnp.uint16)
    return ((ba.astype(jnp.uint32) << 16) | bb.astype(jnp.uint32)).astype(
        jnp.int32)


def _tc_tables(f_ref, feat_ref, w_in_ref, b_in_ref, comp_ref, bases_ref,
               loop_w_ref, conv_b_ref, fcb_ref,
               tabs_ref, fp_ref, cp_ref, dense_ref):
    f = f_ref[...]
    w_aug = jnp.concatenate([w_in_ref[...], b_in_ref[...][None]], axis=0)
    cb0 = w_aug @ bases_ref[0]
    cb1 = w_aug @ bases_ref[1]
    lw = w_aug @ loop_w_ref[...]
    dims = (((1,), (1,)), ((), ()))
    g0 = lax.dot_general(f, cb0, dims, preferred_element_type=jnp.float32)
    g1 = lax.dot_general(f, cb1, dims, preferred_element_type=jnp.float32)
    gl = lax.dot_general(f, lw, dims, preferred_element_type=jnp.float32)
    gb = jnp.dot(f, conv_b_ref[...], preferred_element_type=jnp.float32)
    feat = feat_ref[...]
    dense = (jnp.sum(feat * gl[:, 0:2]) + jnp.sum(gl[:, 2]) + jnp.sum(gb)
             + fcb_ref[0, 0])
    tabs_ref[...] = _pack(g0, g1)
    fp_ref[...] = _pack(feat[:, 0:1], feat[:, 1:2])
    cp_ref[...] = _pack(comp_ref[:, 0:1], comp_ref[:, 1:2])
    dense_ref[...] = jnp.reshape(dense, (1, 1))


_sc_mesh = plsc.VectorSubcoreMesh(core_axis_name="c", subcore_axis_name="s")


def _hi(w):
    return plsc.bitcast(w & jnp.int32(-65536), jnp.float32)


def _lo(w):
    return plsc.bitcast(w << 16, jnp.float32)


@functools.partial(
    pl.kernel,
    out_type=jax.ShapeDtypeStruct((NW, 16), jnp.float32),
    mesh=_sc_mesh,
    compiler_params=pltpu.CompilerParams(
        needs_layout_passes=False, disable_bounds_checks=True),
    scratch_types=[
        pltpu.VMEM((N * 3,), jnp.int32),
        pltpu.VMEM((N,), jnp.int32),
        pltpu.VMEM((8,), jnp.int32),
        pltpu.VMEM((EPW,), jnp.int32),
        pltpu.VMEM((16,), jnp.float32),
        pltpu.SemaphoreType.DMA,
    ],
)
def _sc_edges(tabs_hbm, fp_hbm, cp_hbm, ep_hbm,
              out_hbm, tabs_v, fp_v, cp_v, ep_v, out_v, sem):
    wid = lax.axis_index("s") * NC + lax.axis_index("c")
    base = wid * EPW
    copies = [
        pltpu.make_async_copy(tabs_hbm, tabs_v, sem),
        pltpu.make_async_copy(fp_hbm, fp_v, sem),
        pltpu.make_async_copy(cp_hbm, cp_v, sem),
        pltpu.make_async_copy(ep_hbm.at[pl.ds(base, EPW)], ep_v, sem),
    ]
    for c in copies:
        c.start()
    for c in copies:
        c.wait()

    @plsc.parallel_loop(0, ITERS, unroll=8,
                        carry=jnp.zeros((16,), jnp.float32))
    def acc(i, acc):
        ep = ep_v[pl.ds(i * 16, 16)]
        d3 = lax.shift_right_logical(ep, 17)
        s = (ep >> 3) & jnp.int32(0x3FFF)
        t = ep & jnp.int32(7)
        w0 = plsc.load_gather(tabs_v, [d3])
        w1 = plsc.load_gather(tabs_v, [d3 + 1])
        w2 = plsc.load_gather(tabs_v, [d3 + 2])
        wf = plsc.load_gather(fp_v, [s])
        wc = plsc.load_gather(cp_v, [t])
        fx, fy = _hi(wf), _lo(wf)
        e = (_hi(wc) * (fx * _hi(w0) + fy * _hi(w1) + _hi(w2))
             + _lo(wc) * (fx * _lo(w0) + fy * _lo(w1) + _lo(w2)))
        return acc + e

    out_v[...] = acc
    pltpu.sync_copy(out_v, out_hbm.at[wid])


def kernel(features, edge_index, edge_type, W_in, b_in, comp, bases,
           loop_w, conv_b, fc_w, fc_b):
    F = fc_w.reshape(N, H)
    tabs, fp, cp, dense = pl.pallas_call(
        _tc_tables,
        out_shape=[
            jax.ShapeDtypeStruct((N, 3), jnp.int32),
            jax.ShapeDtypeStruct((N, 1), jnp.int32),
            jax.ShapeDtypeStruct((8, 1), jnp.int32),
            jax.ShapeDtypeStruct((1, 1), jnp.float32),
        ],
    )(F, features, W_in, b_in, comp, bases, loop_w, conv_b, fc_b.reshape(1, 1))

    src32 = edge_index[0].astype(jnp.uint32)
    dst32 = edge_index[1].astype(jnp.uint32)
    ep = ((dst32 * 3) << 17) | (src32 << 3) | edge_type.astype(jnp.uint32)
    ep = lax.bitcast_convert_type(ep, jnp.int32)

    partials = _sc_edges(tabs.reshape(N * 3), fp.reshape(N), cp.reshape(8),
                         ep)
    total = jnp.sum(partials) + dense[0, 0]
    return jax.nn.sigmoid(total).reshape(1, 1)


# speedup vs baseline: 56.5877x; 1.4296x over previous
import functools

import jax
import jax.numpy as jnp
from jax import lax
from jax.experimental import pallas as pl
from jax.experimental.pallas import tpu as pltpu
from jax.experimental.pallas import tpu_sc as plsc

N = 10000
E = 320000
H = 128
NC = 2
NS = 16
NW = NC * NS
EPW = E // NW
ITERS = EPW // 16


def _pack(a, b):
    ba = lax.bitcast_convert_type(a.astype(jnp.bfloat16), jnp.uint16)
    bb = lax.bitcast_convert_type(b.astype(jnp.bfloat16), jnp.uint16)
    return ((ba.astype(jnp.uint32) << 16) | bb.astype(jnp.uint32)).astype(
        jnp.int32)


def _tc_tables(fcw_ref, ftt_ref, compt_ref, w_in_ref, b_in_ref, bases_ref,
               loop_w_ref, conv_b_ref, fcb_ref,
               t0_ref, t1_ref, t2_ref, fp_ref, cp_ref, dense_ref):
    f = fcw_ref[...].reshape(N, H)
    w_aug = jnp.concatenate([w_in_ref[...], b_in_ref[...][None]], axis=0)
    cb_all = jnp.concatenate([
        w_aug @ bases_ref[0],
        w_aug @ bases_ref[1],
        w_aug @ loop_w_ref[...],
        conv_b_ref[...][None],
    ], axis=0)
    tab = lax.dot_general(cb_all, f, (((1,), (1,)), ((), ())),
                          preferred_element_type=jnp.float32)
    ftt = ftt_ref[...]
    dense = (jnp.sum(ftt * tab[6:8, :]) + jnp.sum(tab[8:10, :])
             + fcb_ref[0, 0])
    w3 = _pack(tab[0:3, :], tab[3:6, :])
    t0_ref[...] = w3[0]
    t1_ref[...] = w3[1]
    t2_ref[...] = w3[2]
    fp_ref[...] = _pack(ftt[0], ftt[1])
    cp_ref[...] = _pack(compt_ref[0], compt_ref[1])
    dense_ref[...] = jnp.reshape(dense, (1, 1))


_sc_mesh = plsc.VectorSubcoreMesh(core_axis_name="c", subcore_axis_name="s")


def _hi(w):
    return plsc.bitcast(w & jnp.int32(-65536), jnp.float32)


def _lo(w):
    return plsc.bitcast(w << 16, jnp.float32)


@functools.partial(
    pl.kernel,
    out_type=jax.ShapeDtypeStruct((NW, 16), jnp.float32),
    mesh=_sc_mesh,
    compiler_params=pltpu.CompilerParams(
        needs_layout_passes=False, disable_bounds_checks=True),
    scratch_types=[
        pltpu.VMEM((N,), jnp.int32),
        pltpu.VMEM((N,), jnp.int32),
        pltpu.VMEM((N,), jnp.int32),
        pltpu.VMEM((N,), jnp.int32),
        pltpu.VMEM((8,), jnp.int32),
        pltpu.VMEM((EPW,), jnp.int32),
        pltpu.VMEM((EPW,), jnp.int32),
        pltpu.VMEM((EPW,), jnp.int32),
        pltpu.VMEM((16,), jnp.float32),
        pltpu.SemaphoreType.DMA,
    ],
)
def _sc_edges(t0_hbm, t1_hbm, t2_hbm, fp_hbm, cp_hbm, src_hbm, dst_hbm,
              typ_hbm, out_hbm, t0_v, t1_v, t2_v, fp_v, cp_v, src_v, dst_v,
              typ_v, out_v, sem):
    wid = lax.axis_index("s") * NC + lax.axis_index("c")
    base = wid * EPW
    copies = [
        pltpu.make_async_copy(t0_hbm, t0_v, sem),
        pltpu.make_async_copy(t1_hbm, t1_v, sem),
        pltpu.make_async_copy(t2_hbm, t2_v, sem),
        pltpu.make_async_copy(fp_hbm, fp_v, sem),
        pltpu.make_async_copy(cp_hbm, cp_v, sem),
        pltpu.make_async_copy(src_hbm.at[pl.ds(base, EPW)], src_v, sem),
        pltpu.make_async_copy(dst_hbm.at[pl.ds(base, EPW)], dst_v, sem),
        pltpu.make_async_copy(typ_hbm.at[pl.ds(base, EPW)], typ_v, sem),
    ]
    for c in copies:
        c.start()
    for c in copies:
        c.wait()

    @plsc.parallel_loop(0, ITERS, unroll=8,
                        carry=jnp.zeros((16,), jnp.float32))
    def acc(i, acc):
        s = src_v[pl.ds(i * 16, 16)]
        d = dst_v[pl.ds(i * 16, 16)]
        t = typ_v[pl.ds(i * 16, 16)]
        w0 = plsc.load_gather(t0_v, [d])
        w1 = plsc.load_gather(t1_v, [d])
        w2 = plsc.load_gather(t2_v, [d])
        wf = plsc.load_gather(fp_v, [s])
        wc = plsc.load_gather(cp_v, [t])
        fx, fy = _hi(wf), _lo(wf)
        e = (_hi(wc) * (fx * _hi(w0) + fy * _hi(w1) + _hi(w2))
             + _lo(wc) * (fx * _lo(w0) + fy * _lo(w1) + _lo(w2)))
        return acc + e

    out_v[...] = acc
    pltpu.sync_copy(out_v, out_hbm.at[wid])


def kernel(features, edge_index, edge_type, W_in, b_in, comp, bases,
           loop_w, conv_b, fc_w, fc_b):
    t0, t1, t2, fp, cp, dense = pl.pallas_call(
        _tc_tables,
        out_shape=[
            jax.ShapeDtypeStruct((N,), jnp.int32),
            jax.ShapeDtypeStruct((N,), jnp.int32),
            jax.ShapeDtypeStruct((N,), jnp.int32),
            jax.ShapeDtypeStruct((N,), jnp.int32),
            jax.ShapeDtypeStruct((8,), jnp.int32),
            jax.ShapeDtypeStruct((1, 1), jnp.float32),
        ],
    )(fc_w.reshape(N * H), features.T, comp.T, W_in, b_in, bases, loop_w,
      conv_b, fc_b.reshape(1, 1))

    partials = _sc_edges(t0, t1, t2, fp, cp, edge_index[0], edge_index[1],
                         edge_type)
    total = jnp.sum(partials) + dense[0, 0]
    return jax.nn.sigmoid(total).reshape(1, 1)


# speedup vs baseline: 77.0369x; 1.3614x over previous
import functools

import jax
import jax.numpy as jnp
from jax import lax
from jax.experimental import pallas as pl
from jax.experimental.pallas import tpu as pltpu
from jax.experimental.pallas import tpu_sc as plsc

N = 10000
E = 320000
H = 128
NC = 2
NS = 16
NW = NC * NS
EPW = E // NW
ITERS = EPW // 16


def _pack(a, b):
    ba = lax.bitcast_convert_type(a.astype(jnp.bfloat16), jnp.uint16)
    bb = lax.bitcast_convert_type(b.astype(jnp.bfloat16), jnp.uint16)
    return ((ba.astype(jnp.uint32) << 16) | bb.astype(jnp.uint32)).astype(
        jnp.int32)


def _tc_tables(fcw_ref, ftt_ref, compt_ref, w_in_ref, b_in_ref, bases_ref,
               loop_w_ref, conv_b_ref, fcb_ref, ei_ref,
               t0_ref, t1_ref, t2_ref, fp_ref, cp_ref, dense_ref,
               src_ref, dst_ref):
    ei = ei_ref[...]
    src_ref[...] = ei[0]
    dst_ref[...] = ei[1]
    f = fcw_ref[...].reshape(N, H)
    w_aug = jnp.concatenate([w_in_ref[...], b_in_ref[...][None]], axis=0)
    cb_all = jnp.concatenate([
        w_aug @ bases_ref[0],
        w_aug @ bases_ref[1],
        w_aug @ loop_w_ref[...],
        conv_b_ref[...][None],
    ], axis=0)
    tab = lax.dot_general(cb_all, f, (((1,), (1,)), ((), ())),
                          preferred_element_type=jnp.float32)
    ftt = ftt_ref[...]
    dense = (jnp.sum(ftt * tab[6:8, :]) + jnp.sum(tab[8:10, :])
             + fcb_ref[0, 0])
    w3 = _pack(tab[0:3, :], tab[3:6, :])
    t0_ref[...] = w3[0]
    t1_ref[...] = w3[1]
    t2_ref[...] = w3[2]
    fp_ref[...] = _pack(ftt[0], ftt[1])
    cp_ref[...] = _pack(compt_ref[0], compt_ref[1])
    dense_ref[...] = jnp.reshape(dense, (1, 1))


_sc_mesh = plsc.VectorSubcoreMesh(core_axis_name="c", subcore_axis_name="s")


def _hi(w):
    return plsc.bitcast(w & jnp.int32(-65536), jnp.float32)


def _lo(w):
    return plsc.bitcast(w << 16, jnp.float32)


@functools.partial(
    pl.kernel,
    out_type=jax.ShapeDtypeStruct((NW, 16), jnp.float32),
    mesh=_sc_mesh,
    compiler_params=pltpu.CompilerParams(
        needs_layout_passes=False, disable_bounds_checks=True),
    scratch_types=[
        pltpu.VMEM((N,), jnp.int32),
        pltpu.VMEM((N,), jnp.int32),
        pltpu.VMEM((N,), jnp.int32),
        pltpu.VMEM((N,), jnp.int32),
        pltpu.VMEM((8,), jnp.int32),
        pltpu.VMEM((EPW,), jnp.int32),
        pltpu.VMEM((EPW,), jnp.int32),
        pltpu.VMEM((EPW,), jnp.int32),
        pltpu.VMEM((16,), jnp.float32),
        pltpu.SemaphoreType.DMA,
    ],
)
def _sc_edges(t0_hbm, t1_hbm, t2_hbm, fp_hbm, cp_hbm, src_hbm, dst_hbm,
              typ_hbm, out_hbm, t0_v, t1_v, t2_v, fp_v, cp_v, src_v, dst_v,
              typ_v, out_v, sem):
    wid = lax.axis_index("s") * NC + lax.axis_index("c")
    base = wid * EPW
    copies = [
        pltpu.make_async_copy(t0_hbm, t0_v, sem),
        pltpu.make_async_copy(t1_hbm, t1_v, sem),
        pltpu.make_async_copy(t2_hbm, t2_v, sem),
        pltpu.make_async_copy(fp_hbm, fp_v, sem),
        pltpu.make_async_copy(cp_hbm, cp_v, sem),
        pltpu.make_async_copy(src_hbm.at[pl.ds(base, EPW)], src_v, sem),
        pltpu.make_async_copy(dst_hbm.at[pl.ds(base, EPW)], dst_v, sem),
        pltpu.make_async_copy(typ_hbm.at[pl.ds(base, EPW)], typ_v, sem),
    ]
    for c in copies:
        c.start()
    for c in copies:
        c.wait()

    @plsc.parallel_loop(0, ITERS, unroll=8,
                        carry=jnp.zeros((16,), jnp.float32))
    def acc(i, acc):
        s = src_v[pl.ds(i * 16, 16)]
        d = dst_v[pl.ds(i * 16, 16)]
        t = typ_v[pl.ds(i * 16, 16)]
        w0 = plsc.load_gather(t0_v, [d])
        w1 = plsc.load_gather(t1_v, [d])
        w2 = plsc.load_gather(t2_v, [d])
        wf = plsc.load_gather(fp_v, [s])
        wc = plsc.load_gather(cp_v, [t])
        fx, fy = _hi(wf), _lo(wf)
        e = (_hi(wc) * (fx * _hi(w0) + fy * _hi(w1) + _hi(w2))
             + _lo(wc) * (fx * _lo(w0) + fy * _lo(w1) + _lo(w2)))
        return acc + e

    out_v[...] = acc
    pltpu.sync_copy(out_v, out_hbm.at[wid])


def kernel(features, edge_index, edge_type, W_in, b_in, comp, bases,
           loop_w, conv_b, fc_w, fc_b):
    t0, t1, t2, fp, cp, dense, src, dst = pl.pallas_call(
        _tc_tables,
        out_shape=[
            jax.ShapeDtypeStruct((N,), jnp.int32),
            jax.ShapeDtypeStruct((N,), jnp.int32),
            jax.ShapeDtypeStruct((N,), jnp.int32),
            jax.ShapeDtypeStruct((N,), jnp.int32),
            jax.ShapeDtypeStruct((8,), jnp.int32),
            jax.ShapeDtypeStruct((1, 1), jnp.float32),
            jax.ShapeDtypeStruct((E,), jnp.int32),
            jax.ShapeDtypeStruct((E,), jnp.int32),
        ],
    )(fc_w.reshape(N * H), features.T, comp.T, W_in, b_in, bases, loop_w,
      conv_b, fc_b.reshape(1, 1), edge_index)

    partials = _sc_edges(t0, t1, t2, fp, cp, src, dst, edge_type)
    total = jnp.sum(partials) + dense[0, 0]
    return jax.nn.sigmoid(total).reshape(1, 1)


# speedup vs baseline: 77.1930x; 1.0020x over previous
import functools

import jax
import jax.numpy as jnp
from jax import lax
from jax.experimental import pallas as pl
from jax.experimental.pallas import tpu as pltpu
from jax.experimental.pallas import tpu_sc as plsc

N = 10000
E = 320000
H = 128
NC = 2
NS = 16
NW = NC * NS
EPW = E // NW
ITERS = EPW // 16


def _pack(a, b):
    ba = lax.bitcast_convert_type(a.astype(jnp.bfloat16), jnp.uint16)
    bb = lax.bitcast_convert_type(b.astype(jnp.bfloat16), jnp.uint16)
    return ((ba.astype(jnp.uint32) << 16) | bb.astype(jnp.uint32)).astype(
        jnp.int32)


def _tc_tables(fcw_ref, ftt_ref, compt_ref, w_in_ref, b_in_ref, bases_ref,
               loop_w_ref, conv_b_ref, fcb_ref, ei_ref, typ_ref,
               t0_ref, t1_ref, t2_ref, fp_ref, cp_ref, dense_ref,
               ep_ref):
    ei = ei_ref[...]
    ep_ref[...] = (ei[0] << 17) | (ei[1] << 3) | typ_ref[...]
    f = fcw_ref[...].reshape(N, H)
    w_aug = jnp.concatenate([w_in_ref[...], b_in_ref[...][None]], axis=0)
    cb_all = jnp.concatenate([
        w_aug @ bases_ref[0],
        w_aug @ bases_ref[1],
        w_aug @ loop_w_ref[...],
        conv_b_ref[...][None],
    ], axis=0)
    tab = lax.dot_general(cb_all, f, (((1,), (1,)), ((), ())),
                          preferred_element_type=jnp.float32)
    ftt = ftt_ref[...]
    dense = (jnp.sum(ftt * tab[6:8, :]) + jnp.sum(tab[8:10, :])
             + fcb_ref[0, 0])
    w3 = _pack(tab[0:3, :], tab[3:6, :])
    t0_ref[...] = w3[0]
    t1_ref[...] = w3[1]
    t2_ref[...] = w3[2]
    fp_ref[...] = _pack(ftt[0], ftt[1])
    cp_ref[...] = _pack(compt_ref[0], compt_ref[1])
    dense_ref[...] = jnp.reshape(dense, (1, 1))


_sc_mesh = plsc.VectorSubcoreMesh(core_axis_name="c", subcore_axis_name="s")


def _hi(w):
    return plsc.bitcast(w, jnp.float32)


def _lo(w):
    return plsc.bitcast(w << 16, jnp.float32)


@functools.partial(
    pl.kernel,
    out_type=jax.ShapeDtypeStruct((NW, 16), jnp.float32),
    mesh=_sc_mesh,
    compiler_params=pltpu.CompilerParams(
        needs_layout_passes=False, disable_bounds_checks=True),
    scratch_types=[
        pltpu.VMEM((N,), jnp.int32),
        pltpu.VMEM((N,), jnp.int32),
        pltpu.VMEM((N,), jnp.int32),
        pltpu.VMEM((N,), jnp.int32),
        pltpu.VMEM((8,), jnp.int32),
        pltpu.VMEM((EPW,), jnp.int32),
        pltpu.VMEM((16,), jnp.float32),
        pltpu.SemaphoreType.DMA,
    ],
)
def _sc_edges(t0_hbm, t1_hbm, t2_hbm, fp_hbm, cp_hbm, ep_hbm,
              out_hbm, t0_v, t1_v, t2_v, fp_v, cp_v, ep_v, out_v, sem):
    wid = lax.axis_index("s") * NC + lax.axis_index("c")
    base = wid * EPW
    copies = [
        pltpu.make_async_copy(t0_hbm, t0_v, sem),
        pltpu.make_async_copy(t1_hbm, t1_v, sem),
        pltpu.make_async_copy(t2_hbm, t2_v, sem),
        pltpu.make_async_copy(fp_hbm, fp_v, sem),
        pltpu.make_async_copy(cp_hbm, cp_v, sem),
        pltpu.make_async_copy(ep_hbm.at[pl.ds(base, EPW)], ep_v, sem),
    ]
    for c in copies:
        c.start()
    for c in copies:
        c.wait()

    @plsc.parallel_loop(0, ITERS, unroll=8,
                        carry=jnp.zeros((16,), jnp.float32))
    def acc(i, acc):
        ep = ep_v[pl.ds(i * 16, 16)]
        s = lax.shift_right_logical(ep, 17)
        d = (ep >> 3) & jnp.int32(0x3FFF)
        t = ep & jnp.int32(7)
        w0 = plsc.load_gather(t0_v, [d])
        w1 = plsc.load_gather(t1_v, [d])
        w2 = plsc.load_gather(t2_v, [d])
        wf = plsc.load_gather(fp_v, [s])
        wc = plsc.load_gather(cp_v, [t])
        fx, fy = _hi(wf), _lo(wf)
        e = (_hi(wc) * (fx * _hi(w0) + fy * _hi(w1) + _hi(w2))
             + _lo(wc) * (fx * _lo(w0) + fy * _lo(w1) + _lo(w2)))
        return acc + e

    out_v[...] = acc
    pltpu.sync_copy(out_v, out_hbm.at[wid])


def kernel(features, edge_index, edge_type, W_in, b_in, comp, bases,
           loop_w, conv_b, fc_w, fc_b):
    t0, t1, t2, fp, cp, dense, ep = pl.pallas_call(
        _tc_tables,
        out_shape=[
            jax.ShapeDtypeStruct((N,), jnp.int32),
            jax.ShapeDtypeStruct((N,), jnp.int32),
            jax.ShapeDtypeStruct((N,), jnp.int32),
            jax.ShapeDtypeStruct((N,), jnp.int32),
            jax.ShapeDtypeStruct((8,), jnp.int32),
            jax.ShapeDtypeStruct((1, 1), jnp.float32),
            jax.ShapeDtypeStruct((E,), jnp.int32),
        ],
    )(fc_w.reshape(N * H), features.T, comp.T, W_in, b_in, bases, loop_w,
      conv_b, fc_b.reshape(1, 1), edge_index, edge_type)

    partials = _sc_edges(t0, t1, t2, fp, cp, ep)
    total = jnp.sum(partials) + dense[0, 0]
    return jax.nn.sigmoid(total).reshape(1, 1)
